# baseline (speedup 1.0000x reference)
import functools

import numpy as np
import jax
import jax.numpy as jnp
from jax import lax
from jax.experimental import pallas as pl
from jax.experimental.pallas import tpu as pltpu
from jax.experimental.pallas import tpu_sc as plsc

N = 16384
SQ = 128
_DISTINCT = 64
_MAX_GROUPS = int(1.5 * _DISTINCT)
NGROUPS = min(int(np.random.default_rng(0).integers(1, _MAX_GROUPS + 1)), N)
_BASE, _REM = divmod(N, NGROUPS)

_M32 = 0xFFFFFFFF
_ROTS = ((13, 15, 26, 6), (17, 29, 16, 24))


def _tf2x32_host(k0, k1, x0, x1):
    ks = (k0, k1, (k0 ^ k1 ^ 0x1BD11BDA) & _M32)
    x0 = (x0 + ks[0]) & _M32
    x1 = (x1 + ks[1]) & _M32
    for i in range(5):
        for r in _ROTS[i % 2]:
            x0 = (x0 + x1) & _M32
            x1 = ((x1 << r) | (x1 >> (32 - r))) & _M32
            x1 ^= x0
        x0 = (x0 + ks[(i + 1) % 3]) & _M32
        x1 = (x1 + ks[(i + 2) % 3] + i + 1) & _M32
    return x0, x1


def _derive_round_keys():
    key = _tf2x32_host(0, 0, 0, 1)
    out = []
    for _ in range(2):
        new_key = _tf2x32_host(key[0], key[1], 0, 0)
        subkey = _tf2x32_host(key[0], key[1], 0, 1)
        key = new_key
        out.append(subkey)
    return out


_SK1, _SK2 = _derive_round_keys()


def _group_ids():
    sizes = [_BASE + 1] * _REM + [_BASE] * (NGROUPS - _REM)
    return np.repeat(np.arange(NGROUPS), sizes).astype(np.int32)


def _tf2x32_vec(k0, k1, cnt):
    ks = (jnp.uint32(k0), jnp.uint32(k1),
          jnp.uint32((k0 ^ k1 ^ 0x1BD11BDA) & _M32))
    x0 = jnp.full_like(cnt, ks[0])
    x1 = cnt + ks[1]
    for i in range(5):
        for r in _ROTS[i % 2]:
            x0 = x0 + x1
            x1 = (x1 << jnp.uint32(r)) | (x1 >> jnp.uint32(32 - r))
            x1 = x1 ^ x0
        x0 = x0 + ks[(i + 1) % 3]
        x1 = x1 + ks[(i + 2) % 3] + jnp.uint32(i + 1)
    return x0 ^ x1


def _signed_keys(k0, k1, cnt):
    bits = _tf2x32_vec(k0, k1, cnt)
    return lax.bitcast_convert_type(bits ^ jnp.uint32(0x80000000), jnp.int32)


def _bitonic_pass(key, val, row_io, col_io):
    k = 2
    while k <= N:
        if k < SQ:
            up = (col_io & k) == 0
        elif k < N:
            up = (row_io & (k // SQ)) == 0
        else:
            up = None
        d = k // 2
        while d >= 1:
            if d < SQ:
                side = (col_io & d) == 0
                fwd_k = pltpu.roll(key, SQ - d, 1)
                bwd_k = pltpu.roll(key, d, 1)
                fwd_v = pltpu.roll(val, SQ - d, 1)
                bwd_v = pltpu.roll(val, d, 1)
            else:
                e = d // SQ
                side = (row_io & e) == 0
                fwd_k = pltpu.roll(key, SQ - e, 0)
                bwd_k = pltpu.roll(key, e, 0)
                fwd_v = pltpu.roll(val, SQ - e, 0)
                bwd_v = pltpu.roll(val, e, 0)
            pk = jnp.where(side, fwd_k, bwd_k)
            pv = jnp.where(side, fwd_v, bwd_v)
            take_small = side if up is None else (up == side)
            take_partner = (pk < key) == take_small
            key = jnp.where(take_partner, pk, key)
            val = jnp.where(take_partner, pv, val)
            d //= 2
        k *= 2
    return key, val


def _sort_body(g_ref, gid_ref):
    row_io = lax.broadcasted_iota(jnp.int32, (SQ, SQ), 0)
    col_io = lax.broadcasted_iota(jnp.int32, (SQ, SQ), 1)
    cnt = (lax.broadcasted_iota(jnp.uint32, (SQ, SQ), 0) * jnp.uint32(SQ)
           + lax.broadcasted_iota(jnp.uint32, (SQ, SQ), 1))
    val = row_io * SQ + col_io
    key1 = _signed_keys(_SK1[0], _SK1[1], cnt)
    _, val = _bitonic_pass(key1, val, row_io, col_io)
    key2 = _signed_keys(_SK2[0], _SK2[1], cnt)
    _, r = _bitonic_pass(key2, val, row_io, col_io)
    del g_ref
    gid_ref[...] = r


def _sc_scatter():
    info = plsc.get_sparse_core_info()
    nc, ns = 1, info.num_subcores
    nw = nc * ns
    rows_per = SQ // nw
    mesh = plsc.VectorSubcoreMesh(core_axis_name="c", subcore_axis_name="s", num_cores=1)

    @functools.partial(
        pl.kernel,
        out_type=jax.ShapeDtypeStruct((N,), jnp.int32),
        mesh=mesh,
        compiler_params=pltpu.CompilerParams(needs_layout_passes=False),
        scratch_types=[
            pltpu.VMEM((rows_per, SQ), jnp.int32),
            pltpu.VMEM((rows_per, SQ), jnp.int32),
            pltpu.SemaphoreType.DMA,
        ],
    )
    def scat(r_hbm, g_hbm, out_hbm, r_v, g_v, sem):
        wid = lax.axis_index("s") * nc + lax.axis_index("c")
        base = wid * rows_per
        pltpu.sync_copy(r_hbm.at[pl.ds(base, rows_per), :], r_v)
        pltpu.sync_copy(g_hbm.at[pl.ds(base, rows_per), :], g_v)
        copies = [
            pltpu.async_copy(g_v.at[q], out_hbm.at[r_v.at[q]], sem)
            for q in range(rows_per)
        ]
        for c in copies:
            c.wait()

    return scat


_OH_BLK = 2048
_OH_GRID = N // _OH_BLK


def _onehot_body(j_ref, o_ref):
    jrow = j_ref[0]
    gio = lax.broadcasted_iota(jnp.int32, (NGROUPS, _OH_BLK), 0)
    o_ref[...] = (jrow == gio).astype(jnp.float32)


def kernel(x):
    del x
    g_const = jnp.asarray(_group_ids().reshape(SQ, SQ))
    r = pl.pallas_call(
        _sort_body,
        out_shape=jax.ShapeDtypeStruct((SQ, SQ), jnp.int32),
    )(g_const)
    gid = _sc_scatter()(r, g_const)

    masks = pl.pallas_call(
        _onehot_body,
        grid=(_OH_GRID,),
        in_specs=[pl.BlockSpec((1, 1, _OH_BLK), lambda g: (g, 0, 0))],
        out_specs=pl.BlockSpec((NGROUPS, _OH_BLK), lambda g: (0, g)),
        out_shape=jax.ShapeDtypeStruct((NGROUPS, N), jnp.float32),
    )(gid.reshape(_OH_GRID, 1, _OH_BLK))
    return masks


# speedup vs baseline: 4.2303x; 4.2303x over previous
import functools

import numpy as np
import jax
import jax.numpy as jnp
from jax import lax
from jax.experimental import pallas as pl
from jax.experimental.pallas import tpu as pltpu
from jax.experimental.pallas import tpu_sc as plsc

N = 16384
SQ = 128
_DISTINCT = 64
_MAX_GROUPS = int(1.5 * _DISTINCT)
NGROUPS = min(int(np.random.default_rng(0).integers(1, _MAX_GROUPS + 1)), N)
_BASE, _REM = divmod(N, NGROUPS)

_M32 = 0xFFFFFFFF
_ROTS = ((13, 15, 26, 6), (17, 29, 16, 24))


def _tf2x32_host(k0, k1, x0, x1):
    ks = (k0, k1, (k0 ^ k1 ^ 0x1BD11BDA) & _M32)
    x0 = (x0 + ks[0]) & _M32
    x1 = (x1 + ks[1]) & _M32
    for i in range(5):
        for r in _ROTS[i % 2]:
            x0 = (x0 + x1) & _M32
            x1 = ((x1 << r) | (x1 >> (32 - r))) & _M32
            x1 ^= x0
        x0 = (x0 + ks[(i + 1) % 3]) & _M32
        x1 = (x1 + ks[(i + 2) % 3] + i + 1) & _M32
    return x0, x1


def _derive_round_keys():
    key = _tf2x32_host(0, 0, 0, 1)
    out = []
    for _ in range(2):
        new_key = _tf2x32_host(key[0], key[1], 0, 0)
        subkey = _tf2x32_host(key[0], key[1], 0, 1)
        key = new_key
        out.append(subkey)
    return out


_SK1, _SK2 = _derive_round_keys()


def _group_ids():
    sizes = [_BASE + 1] * _REM + [_BASE] * (NGROUPS - _REM)
    return np.repeat(np.arange(NGROUPS), sizes).astype(np.int32)


def _tf2x32_vec(k0, k1, cnt):
    ks = (jnp.uint32(k0), jnp.uint32(k1),
          jnp.uint32((k0 ^ k1 ^ 0x1BD11BDA) & _M32))
    x0 = jnp.full_like(cnt, ks[0])
    x1 = cnt + ks[1]
    for i in range(5):
        for r in _ROTS[i % 2]:
            x0 = x0 + x1
            x1 = (x1 << jnp.uint32(r)) | (x1 >> jnp.uint32(32 - r))
            x1 = x1 ^ x0
        x0 = x0 + ks[(i + 1) % 3]
        x1 = x1 + ks[(i + 2) % 3] + jnp.uint32(i + 1)
    return x0 ^ x1


def _signed_keys(k0, k1, cnt):
    bits = _tf2x32_vec(k0, k1, cnt)
    return lax.bitcast_convert_type(bits ^ jnp.uint32(0x80000000), jnp.int32)


_NSLAB = 8
_SROWS = SQ // _NSLAB


def _bitonic_pass(keys, vals, t_io, col_io):
    has_val = vals is not None

    def up_of(k, s):
        if k < _SROWS:
            return (t_io & k) == 0
        if k < SQ:
            return ((s * _SROWS) & k) == 0
        if k == N:
            return True
        return (col_io & (k // SQ)) == 0

    k = 2
    while k <= N:
        d = k // 2
        while d >= 1:
            if _SROWS <= d < SQ:
                x = d // _SROWS
                for a in range(_NSLAB):
                    if a & x:
                        continue
                    b = a ^ x
                    up = up_of(k, a)
                    c = keys[b] < keys[a]
                    if isinstance(up, bool):
                        m = c if up else ~c
                    else:
                        m = c == up
                    ka = jnp.where(m, keys[b], keys[a])
                    kb = jnp.where(m, keys[a], keys[b])
                    keys[a], keys[b] = ka, kb
                    if has_val:
                        va = jnp.where(m, vals[b], vals[a])
                        vb = jnp.where(m, vals[a], vals[b])
                        vals[a], vals[b] = va, vb
            else:
                if d < _SROWS:
                    side = (t_io & d) == 0
                else:
                    dd = d // SQ
                    side = (col_io & dd) == 0
                for s in range(_NSLAB):
                    key = keys[s]
                    if d < _SROWS:
                        fwd_k = pltpu.roll(key, _SROWS - d, 0)
                        bwd_k = pltpu.roll(key, d, 0)
                    else:
                        dd = d // SQ
                        fwd_k = pltpu.roll(key, SQ - dd, 1)
                        bwd_k = pltpu.roll(key, dd, 1)
                    pk = jnp.where(side, fwd_k, bwd_k)
                    up = up_of(k, s)
                    if up is True:
                        tp = (pk < key) == side
                    elif up is False:
                        tp = (pk < key) != side
                    else:
                        tp = (pk < key) == (up == side)
                    keys[s] = jnp.where(tp, pk, key)
                    if has_val:
                        val = vals[s]
                        if d < _SROWS:
                            fwd_v = pltpu.roll(val, _SROWS - d, 0)
                            bwd_v = pltpu.roll(val, d, 0)
                        else:
                            dd = d // SQ
                            fwd_v = pltpu.roll(val, SQ - dd, 1)
                            bwd_v = pltpu.roll(val, dd, 1)
                        pv = jnp.where(side, fwd_v, bwd_v)
                        vals[s] = jnp.where(tp, pv, val)
            d //= 2
        k *= 2
    return keys, vals


def _sort_body(gt_ref, gid_ref):
    t_io = lax.broadcasted_iota(jnp.int32, (_SROWS, 1), 0)
    col_io = lax.broadcasted_iota(jnp.int32, (1, SQ), 1)
    t_u = lax.broadcasted_iota(jnp.uint32, (_SROWS, SQ), 0)
    c_u = lax.broadcasted_iota(jnp.uint32, (_SROWS, SQ), 1)
    cnts = [c_u * jnp.uint32(SQ) + t_u + jnp.uint32(s * _SROWS)
            for s in range(_NSLAB)]
    vals = [col_io * SQ + t_io + s * _SROWS for s in range(_NSLAB)]
    keys1 = [_signed_keys(_SK1[0], _SK1[1], c) for c in cnts]
    _, vals = _bitonic_pass(keys1, vals, t_io, col_io)
    keys2 = [_signed_keys(_SK2[0], _SK2[1], c) for c in cnts]
    _, rs = _bitonic_pass(keys2, vals, t_io, col_io)
    packed = [rs[s] * SQ + gt_ref[s * _SROWS:(s + 1) * _SROWS, :]
              for s in range(_NSLAB)]
    packed, _ = _bitonic_pass(packed, None, t_io, col_io)
    gid_t = jnp.concatenate([p & (SQ - 1) for p in packed], axis=0)
    gid_ref[...] = gid_t.T


def _sc_scatter():
    info = plsc.get_sparse_core_info()
    nc, ns = info.num_cores, info.num_subcores
    nw = nc * ns
    rows_per = SQ // nw
    mesh = plsc.VectorSubcoreMesh(core_axis_name="c", subcore_axis_name="s")

    @functools.partial(
        pl.kernel,
        out_type=jax.ShapeDtypeStruct((N,), jnp.int32),
        mesh=mesh,
        compiler_params=pltpu.CompilerParams(needs_layout_passes=False),
        scratch_types=[
            pltpu.VMEM((rows_per, SQ), jnp.int32),
            pltpu.VMEM((rows_per, SQ), jnp.int32),
            pltpu.SemaphoreType.DMA,
        ],
    )
    def scat(r_hbm, g_hbm, out_hbm, r_v, g_v, sem):
        wid = lax.axis_index("s") * nc + lax.axis_index("c")
        base = wid * rows_per
        pltpu.sync_copy(r_hbm.at[pl.ds(base, rows_per), :], r_v)
        pltpu.sync_copy(g_hbm.at[pl.ds(base, rows_per), :], g_v)
        copies = [
            pltpu.async_copy(g_v.at[q], out_hbm.at[r_v.at[q]], sem)
            for q in range(rows_per)
        ]
        for c in copies:
            c.wait()

    return scat


_OH_BLK = 2048
_OH_GRID = N // _OH_BLK


def _onehot_body(j_ref, o_ref):
    jrow = j_ref[0]
    gio = lax.broadcasted_iota(jnp.int32, (NGROUPS, _OH_BLK), 0)
    o_ref[...] = (jrow == gio).astype(jnp.float32)


def kernel(x):
    del x
    g_const = jnp.asarray(np.ascontiguousarray(_group_ids().reshape(SQ, SQ).T))
    gid = pl.pallas_call(
        _sort_body,
        out_shape=jax.ShapeDtypeStruct((SQ, SQ), jnp.int32),
    )(g_const)

    masks = pl.pallas_call(
        _onehot_body,
        grid=(_OH_GRID,),
        in_specs=[pl.BlockSpec((1, 1, _OH_BLK), lambda g: (g, 0, 0))],
        out_specs=pl.BlockSpec((NGROUPS, _OH_BLK), lambda g: (0, g)),
        out_shape=jax.ShapeDtypeStruct((NGROUPS, N), jnp.float32),
    )(gid.reshape(_OH_GRID, 1, _OH_BLK))
    return masks


# speedup vs baseline: 4.7077x; 1.1129x over previous
import functools

import numpy as np
import jax
import jax.numpy as jnp
from jax import lax
from jax.experimental import pallas as pl
from jax.experimental.pallas import tpu as pltpu
from jax.experimental.pallas import tpu_sc as plsc

N = 16384
SQ = 128
_DISTINCT = 64
_MAX_GROUPS = int(1.5 * _DISTINCT)
NGROUPS = min(int(np.random.default_rng(0).integers(1, _MAX_GROUPS + 1)), N)
_BASE, _REM = divmod(N, NGROUPS)

_M32 = 0xFFFFFFFF
_ROTS = ((13, 15, 26, 6), (17, 29, 16, 24))


def _tf2x32_host(k0, k1, x0, x1):
    ks = (k0, k1, (k0 ^ k1 ^ 0x1BD11BDA) & _M32)
    x0 = (x0 + ks[0]) & _M32
    x1 = (x1 + ks[1]) & _M32
    for i in range(5):
        for r in _ROTS[i % 2]:
            x0 = (x0 + x1) & _M32
            x1 = ((x1 << r) | (x1 >> (32 - r))) & _M32
            x1 ^= x0
        x0 = (x0 + ks[(i + 1) % 3]) & _M32
        x1 = (x1 + ks[(i + 2) % 3] + i + 1) & _M32
    return x0, x1


def _derive_round_keys():
    key = _tf2x32_host(0, 0, 0, 1)
    out = []
    for _ in range(2):
        new_key = _tf2x32_host(key[0], key[1], 0, 0)
        subkey = _tf2x32_host(key[0], key[1], 0, 1)
        key = new_key
        out.append(subkey)
    return out


_SK1, _SK2 = _derive_round_keys()


def _group_ids():
    sizes = [_BASE + 1] * _REM + [_BASE] * (NGROUPS - _REM)
    return np.repeat(np.arange(NGROUPS), sizes).astype(np.int32)


def _tf2x32_vec(k0, k1, cnt):
    ks = (jnp.uint32(k0), jnp.uint32(k1),
          jnp.uint32((k0 ^ k1 ^ 0x1BD11BDA) & _M32))
    x0 = jnp.full_like(cnt, ks[0])
    x1 = cnt + ks[1]
    for i in range(5):
        for r in _ROTS[i % 2]:
            x0 = x0 + x1
            x1 = (x1 << jnp.uint32(r)) | (x1 >> jnp.uint32(32 - r))
            x1 = x1 ^ x0
        x0 = x0 + ks[(i + 1) % 3]
        x1 = x1 + ks[(i + 2) % 3] + jnp.uint32(i + 1)
    return x0 ^ x1


def _signed_keys(k0, k1, cnt):
    bits = _tf2x32_vec(k0, k1, cnt)
    return lax.bitcast_convert_type(bits ^ jnp.uint32(0x80000000), jnp.int32)


_NSLAB = 16
_SROWS = SQ // _NSLAB


def _bitonic_pass(keys, vals, t_io, col_io):
    has_val = vals is not None

    def up_of(k, s):
        if k < _SROWS:
            return (t_io & k) == 0
        if k < SQ:
            return ((s * _SROWS) & k) == 0
        if k == N:
            return True
        return (col_io & (k // SQ)) == 0

    k = 2
    while k <= N:
        d = k // 2
        while d >= 1:
            if _SROWS <= d < SQ:
                x = d // _SROWS
                for a in range(_NSLAB):
                    if a & x:
                        continue
                    b = a ^ x
                    up = up_of(k, a)
                    c = keys[b] < keys[a]
                    if isinstance(up, bool):
                        m = c if up else ~c
                    else:
                        m = c == up
                    ka = jnp.where(m, keys[b], keys[a])
                    kb = jnp.where(m, keys[a], keys[b])
                    keys[a], keys[b] = ka, kb
                    if has_val:
                        va = jnp.where(m, vals[b], vals[a])
                        vb = jnp.where(m, vals[a], vals[b])
                        vals[a], vals[b] = va, vb
            else:
                if d < _SROWS:
                    side = (t_io & d) == 0
                else:
                    dd = d // SQ
                    side = (col_io & dd) == 0
                for s in range(_NSLAB):
                    key = keys[s]
                    if d < _SROWS:
                        fwd_k = pltpu.roll(key, _SROWS - d, 0)
                        bwd_k = pltpu.roll(key, d, 0)
                    else:
                        dd = d // SQ
                        fwd_k = pltpu.roll(key, SQ - dd, 1)
                        bwd_k = pltpu.roll(key, dd, 1)
                    pk = jnp.where(side, fwd_k, bwd_k)
                    up = up_of(k, s)
                    if up is True:
                        tp = (pk < key) == side
                    elif up is False:
                        tp = (pk < key) != side
                    else:
                        tp = (pk < key) == (up == side)
                    keys[s] = jnp.where(tp, pk, key)
                    if has_val:
                        val = vals[s]
                        if d < _SROWS:
                            fwd_v = pltpu.roll(val, _SROWS - d, 0)
                            bwd_v = pltpu.roll(val, d, 0)
                        else:
                            dd = d // SQ
                            fwd_v = pltpu.roll(val, SQ - dd, 1)
                            bwd_v = pltpu.roll(val, dd, 1)
                        pv = jnp.where(side, fwd_v, bwd_v)
                        vals[s] = jnp.where(tp, pv, val)
            d //= 2
        k *= 2
    return keys, vals


def _sort_body(gt_ref, gid_ref):
    t_io = lax.broadcasted_iota(jnp.int32, (_SROWS, 1), 0)
    col_io = lax.broadcasted_iota(jnp.int32, (1, SQ), 1)
    t_u = lax.broadcasted_iota(jnp.uint32, (_SROWS, SQ), 0)
    c_u = lax.broadcasted_iota(jnp.uint32, (_SROWS, SQ), 1)
    cnts = [c_u * jnp.uint32(SQ) + t_u + jnp.uint32(s * _SROWS)
            for s in range(_NSLAB)]
    vals = [col_io * SQ + t_io + s * _SROWS for s in range(_NSLAB)]
    keys1 = [_signed_keys(_SK1[0], _SK1[1], c) for c in cnts]
    _, vals = _bitonic_pass(keys1, vals, t_io, col_io)
    keys2 = [_signed_keys(_SK2[0], _SK2[1], c) for c in cnts]
    _, rs = _bitonic_pass(keys2, vals, t_io, col_io)
    packed = [rs[s] * SQ + gt_ref[s * _SROWS:(s + 1) * _SROWS, :]
              for s in range(_NSLAB)]
    packed, _ = _bitonic_pass(packed, None, t_io, col_io)
    gid_t = jnp.concatenate([p & (SQ - 1) for p in packed], axis=0)
    gid_ref[...] = gid_t.T


def _sc_scatter():
    info = plsc.get_sparse_core_info()
    nc, ns = info.num_cores, info.num_subcores
    nw = nc * ns
    rows_per = SQ // nw
    mesh = plsc.VectorSubcoreMesh(core_axis_name="c", subcore_axis_name="s")

    @functools.partial(
        pl.kernel,
        out_type=jax.ShapeDtypeStruct((N,), jnp.int32),
        mesh=mesh,
        compiler_params=pltpu.CompilerParams(needs_layout_passes=False),
        scratch_types=[
            pltpu.VMEM((rows_per, SQ), jnp.int32),
            pltpu.VMEM((rows_per, SQ), jnp.int32),
            pltpu.SemaphoreType.DMA,
        ],
    )
    def scat(r_hbm, g_hbm, out_hbm, r_v, g_v, sem):
        wid = lax.axis_index("s") * nc + lax.axis_index("c")
        base = wid * rows_per
        pltpu.sync_copy(r_hbm.at[pl.ds(base, rows_per), :], r_v)
        pltpu.sync_copy(g_hbm.at[pl.ds(base, rows_per), :], g_v)
        copies = [
            pltpu.async_copy(g_v.at[q], out_hbm.at[r_v.at[q]], sem)
            for q in range(rows_per)
        ]
        for c in copies:
            c.wait()

    return scat


_OH_BLK = 2048
_OH_GRID = N // _OH_BLK


def _onehot_body(j_ref, o_ref):
    jrow = j_ref[0]
    gio = lax.broadcasted_iota(jnp.int32, (NGROUPS, _OH_BLK), 0)
    o_ref[...] = (jrow == gio).astype(jnp.float32)


def kernel(x):
    del x
    g_const = jnp.asarray(np.ascontiguousarray(_group_ids().reshape(SQ, SQ).T))
    gid = pl.pallas_call(
        _sort_body,
        out_shape=jax.ShapeDtypeStruct((SQ, SQ), jnp.int32),
    )(g_const)

    masks = pl.pallas_call(
        _onehot_body,
        grid=(_OH_GRID,),
        in_specs=[pl.BlockSpec((1, 1, _OH_BLK), lambda g: (g, 0, 0))],
        out_specs=pl.BlockSpec((NGROUPS, _OH_BLK), lambda g: (0, g)),
        out_shape=jax.ShapeDtypeStruct((NGROUPS, N), jnp.float32),
    )(gid.reshape(_OH_GRID, 1, _OH_BLK))
    return masks


# speedup vs baseline: 5.7787x; 1.2275x over previous
import functools

import numpy as np
import jax
import jax.numpy as jnp
from jax import lax
from jax.experimental import pallas as pl
from jax.experimental.pallas import tpu as pltpu
from jax.experimental.pallas import tpu_sc as plsc

N = 16384
SQ = 128
_DISTINCT = 64
_MAX_GROUPS = int(1.5 * _DISTINCT)
NGROUPS = min(int(np.random.default_rng(0).integers(1, _MAX_GROUPS + 1)), N)
_BASE, _REM = divmod(N, NGROUPS)

_M32 = 0xFFFFFFFF
_ROTS = ((13, 15, 26, 6), (17, 29, 16, 24))


def _tf2x32_host(k0, k1, x0, x1):
    ks = (k0, k1, (k0 ^ k1 ^ 0x1BD11BDA) & _M32)
    x0 = (x0 + ks[0]) & _M32
    x1 = (x1 + ks[1]) & _M32
    for i in range(5):
        for r in _ROTS[i % 2]:
            x0 = (x0 + x1) & _M32
            x1 = ((x1 << r) | (x1 >> (32 - r))) & _M32
            x1 ^= x0
        x0 = (x0 + ks[(i + 1) % 3]) & _M32
        x1 = (x1 + ks[(i + 2) % 3] + i + 1) & _M32
    return x0, x1


def _derive_round_keys():
    key = _tf2x32_host(0, 0, 0, 1)
    out = []
    for _ in range(2):
        new_key = _tf2x32_host(key[0], key[1], 0, 0)
        subkey = _tf2x32_host(key[0], key[1], 0, 1)
        key = new_key
        out.append(subkey)
    return out


_SK1, _SK2 = _derive_round_keys()


def _group_ids():
    sizes = [_BASE + 1] * _REM + [_BASE] * (NGROUPS - _REM)
    return np.repeat(np.arange(NGROUPS), sizes).astype(np.int32)


def _tf2x32_vec(k0, k1, cnt):
    ks = (jnp.uint32(k0), jnp.uint32(k1),
          jnp.uint32((k0 ^ k1 ^ 0x1BD11BDA) & _M32))
    x0 = jnp.full_like(cnt, ks[0])
    x1 = cnt + ks[1]
    for i in range(5):
        for r in _ROTS[i % 2]:
            x0 = x0 + x1
            x1 = (x1 << jnp.uint32(r)) | (x1 >> jnp.uint32(32 - r))
            x1 = x1 ^ x0
        x0 = x0 + ks[(i + 1) % 3]
        x1 = x1 + ks[(i + 2) % 3] + jnp.uint32(i + 1)
    return x0 ^ x1


def _signed_keys(k0, k1, cnt):
    bits = _tf2x32_vec(k0, k1, cnt)
    return lax.bitcast_convert_type(bits ^ jnp.uint32(0x80000000), jnp.int32)


_NSLAB = 16
_SROWS = SQ // _NSLAB


def _bitonic_pass(keys, vals, t_io, col_io):
    has_val = vals is not None

    def up_of(k, s):
        if k < _SROWS:
            return (t_io & k) == 0
        if k < SQ:
            return ((s * _SROWS) & k) == 0
        if k == N:
            return True
        return (col_io & (k // SQ)) == 0

    k = 2
    while k <= N:
        d = k // 2
        while d >= 1:
            if _SROWS <= d < SQ:
                x = d // _SROWS
                for a in range(_NSLAB):
                    if a & x:
                        continue
                    b = a ^ x
                    up = up_of(k, a)
                    c = keys[b] < keys[a]
                    if isinstance(up, bool):
                        m = c if up else ~c
                    else:
                        m = c == up
                    ka = jnp.where(m, keys[b], keys[a])
                    kb = jnp.where(m, keys[a], keys[b])
                    keys[a], keys[b] = ka, kb
                    if has_val:
                        va = jnp.where(m, vals[b], vals[a])
                        vb = jnp.where(m, vals[a], vals[b])
                        vals[a], vals[b] = va, vb
            else:
                if d < _SROWS:
                    side = (t_io & d) == 0
                else:
                    dd = d // SQ
                    side = (col_io & dd) == 0
                for s in range(_NSLAB):
                    key = keys[s]
                    if d < _SROWS:
                        fwd_k = pltpu.roll(key, _SROWS - d, 0)
                        bwd_k = pltpu.roll(key, d, 0)
                    else:
                        dd = d // SQ
                        fwd_k = pltpu.roll(key, SQ - dd, 1)
                        bwd_k = pltpu.roll(key, dd, 1)
                    pk = jnp.where(side, fwd_k, bwd_k)
                    up = up_of(k, s)
                    if up is True:
                        tp = (pk < key) == side
                    elif up is False:
                        tp = (pk < key) != side
                    else:
                        tp = (pk < key) == (up == side)
                    keys[s] = jnp.where(tp, pk, key)
                    if has_val:
                        val = vals[s]
                        if d < _SROWS:
                            fwd_v = pltpu.roll(val, _SROWS - d, 0)
                            bwd_v = pltpu.roll(val, d, 0)
                        else:
                            dd = d // SQ
                            fwd_v = pltpu.roll(val, SQ - dd, 1)
                            bwd_v = pltpu.roll(val, dd, 1)
                        pv = jnp.where(side, fwd_v, bwd_v)
                        vals[s] = jnp.where(tp, pv, val)
            d //= 2
        k *= 2
    return keys, vals


def _sort_body(gt_ref, out_ref):
    t_io = lax.broadcasted_iota(jnp.int32, (_SROWS, 1), 0)
    col_io = lax.broadcasted_iota(jnp.int32, (1, SQ), 1)
    t_u = lax.broadcasted_iota(jnp.uint32, (_SROWS, SQ), 0)
    c_u = lax.broadcasted_iota(jnp.uint32, (_SROWS, SQ), 1)
    cnts = [c_u * jnp.uint32(SQ) + t_u + jnp.uint32(s * _SROWS)
            for s in range(_NSLAB)]
    vals = [col_io * SQ + t_io + s * _SROWS for s in range(_NSLAB)]
    keys1 = [_signed_keys(_SK1[0], _SK1[1], c) for c in cnts]
    _, vals = _bitonic_pass(keys1, vals, t_io, col_io)
    keys2 = [_signed_keys(_SK2[0], _SK2[1], c) for c in cnts]
    _, rs = _bitonic_pass(keys2, vals, t_io, col_io)
    packed = [rs[s] * SQ + gt_ref[s * _SROWS:(s + 1) * _SROWS, :]
              for s in range(_NSLAB)]
    packed, _ = _bitonic_pass(packed, None, t_io, col_io)
    gid_t = jnp.concatenate([p & (SQ - 1) for p in packed], axis=0)
    gid = gid_t.T
    gio = lax.broadcasted_iota(jnp.int32, (NGROUPS, 1), 0)
    for a in range(SQ):
        row = gid[a:a + 1, :]
        out_ref[:, a * SQ:(a + 1) * SQ] = (row == gio).astype(jnp.float32)


def _sc_scatter():
    info = plsc.get_sparse_core_info()
    nc, ns = info.num_cores, info.num_subcores
    nw = nc * ns
    rows_per = SQ // nw
    mesh = plsc.VectorSubcoreMesh(core_axis_name="c", subcore_axis_name="s")

    @functools.partial(
        pl.kernel,
        out_type=jax.ShapeDtypeStruct((N,), jnp.int32),
        mesh=mesh,
        compiler_params=pltpu.CompilerParams(needs_layout_passes=False),
        scratch_types=[
            pltpu.VMEM((rows_per, SQ), jnp.int32),
            pltpu.VMEM((rows_per, SQ), jnp.int32),
            pltpu.SemaphoreType.DMA,
        ],
    )
    def scat(r_hbm, g_hbm, out_hbm, r_v, g_v, sem):
        wid = lax.axis_index("s") * nc + lax.axis_index("c")
        base = wid * rows_per
        pltpu.sync_copy(r_hbm.at[pl.ds(base, rows_per), :], r_v)
        pltpu.sync_copy(g_hbm.at[pl.ds(base, rows_per), :], g_v)
        copies = [
            pltpu.async_copy(g_v.at[q], out_hbm.at[r_v.at[q]], sem)
            for q in range(rows_per)
        ]
        for c in copies:
            c.wait()

    return scat


_OH_BLK = 2048
_OH_GRID = N // _OH_BLK


def _onehot_body(j_ref, o_ref):
    jrow = j_ref[0]
    gio = lax.broadcasted_iota(jnp.int32, (NGROUPS, _OH_BLK), 0)
    o_ref[...] = (jrow == gio).astype(jnp.float32)


def kernel(x):
    del x
    g_const = jnp.asarray(np.ascontiguousarray(_group_ids().reshape(SQ, SQ).T))
    return pl.pallas_call(
        _sort_body,
        out_shape=jax.ShapeDtypeStruct((NGROUPS, N), jnp.float32),
    )(g_const)


# speedup vs baseline: 5.7968x; 1.0031x over previous
import numpy as np
import jax
import jax.numpy as jnp
from jax import lax
from jax.experimental import pallas as pl
from jax.experimental.pallas import tpu as pltpu

N = 16384
SQ = 128
_DISTINCT = 64
_MAX_GROUPS = int(1.5 * _DISTINCT)
NGROUPS = min(int(np.random.default_rng(0).integers(1, _MAX_GROUPS + 1)), N)
_BASE, _REM = divmod(N, NGROUPS)

_M32 = 0xFFFFFFFF
_ROTS = ((13, 15, 26, 6), (17, 29, 16, 24))


def _tf2x32_host(k0, k1, x0, x1):
    ks = (k0, k1, (k0 ^ k1 ^ 0x1BD11BDA) & _M32)
    x0 = (x0 + ks[0]) & _M32
    x1 = (x1 + ks[1]) & _M32
    for i in range(5):
        for r in _ROTS[i % 2]:
            x0 = (x0 + x1) & _M32
            x1 = ((x1 << r) | (x1 >> (32 - r))) & _M32
            x1 ^= x0
        x0 = (x0 + ks[(i + 1) % 3]) & _M32
        x1 = (x1 + ks[(i + 2) % 3] + i + 1) & _M32
    return x0, x1


def _derive_round_keys():
    key = _tf2x32_host(0, 0, 0, 1)
    out = []
    for _ in range(2):
        new_key = _tf2x32_host(key[0], key[1], 0, 0)
        subkey = _tf2x32_host(key[0], key[1], 0, 1)
        key = new_key
        out.append(subkey)
    return out


_SK1, _SK2 = _derive_round_keys()


def _group_ids():
    sizes = [_BASE + 1] * _REM + [_BASE] * (NGROUPS - _REM)
    return np.repeat(np.arange(NGROUPS), sizes).astype(np.int32)


def _tf2x32_vec(k0, k1, cnt):
    ks = (jnp.uint32(k0), jnp.uint32(k1),
          jnp.uint32((k0 ^ k1 ^ 0x1BD11BDA) & _M32))
    x0 = jnp.full_like(cnt, ks[0])
    x1 = cnt + ks[1]
    for i in range(5):
        for r in _ROTS[i % 2]:
            x0 = x0 + x1
            x1 = (x1 << jnp.uint32(r)) | (x1 >> jnp.uint32(32 - r))
            x1 = x1 ^ x0
        x0 = x0 + ks[(i + 1) % 3]
        x1 = x1 + ks[(i + 2) % 3] + jnp.uint32(i + 1)
    return x0 ^ x1


def _signed_keys(k0, k1, cnt):
    bits = _tf2x32_vec(k0, k1, cnt)
    return lax.bitcast_convert_type(bits ^ jnp.uint32(0x80000000), jnp.int32)


_NSLAB = 16
_SROWS = SQ // _NSLAB


def _bitonic_pass(keys, vals, t_io, col_io):
    has_val = vals is not None

    def up_of(k, s):
        if k < _SROWS:
            return (t_io & k) == 0
        if k < SQ:
            return ((s * _SROWS) & k) == 0
        if k == N:
            return True
        return (col_io & (k // SQ)) == 0

    k = 2
    while k <= N:
        d = k // 2
        while d >= 1:
            if _SROWS <= d < SQ:
                x = d // _SROWS
                for a in range(_NSLAB):
                    if a & x:
                        continue
                    b = a ^ x
                    up = up_of(k, a)
                    c = keys[b] < keys[a]
                    if isinstance(up, bool):
                        m = c if up else ~c
                    else:
                        m = c == up
                    ka = jnp.where(m, keys[b], keys[a])
                    kb = jnp.where(m, keys[a], keys[b])
                    keys[a], keys[b] = ka, kb
                    if has_val:
                        va = jnp.where(m, vals[b], vals[a])
                        vb = jnp.where(m, vals[a], vals[b])
                        vals[a], vals[b] = va, vb
            else:
                if d < _SROWS:
                    side = (t_io & d) == 0
                else:
                    dd = d // SQ
                    side = (col_io & dd) == 0
                up0 = up_of(k, 0)
                ts_mask = None
                if not isinstance(up0, bool):
                    ts_mask = up0 == side
                for s in range(_NSLAB):
                    key = keys[s]
                    if d < _SROWS:
                        fwd_k = pltpu.roll(key, _SROWS - d, 0)
                        bwd_k = pltpu.roll(key, d, 0)
                    else:
                        dd = d // SQ
                        fwd_k = pltpu.roll(key, SQ - dd, 1)
                        bwd_k = pltpu.roll(key, dd, 1)
                    pk = jnp.where(side, fwd_k, bwd_k)
                    up = up_of(k, s)
                    if up is True:
                        tp = (pk < key) == side
                    elif up is False:
                        tp = (pk < key) != side
                    else:
                        tp = (pk < key) == ts_mask
                    keys[s] = jnp.where(tp, pk, key)
                    if has_val:
                        val = vals[s]
                        if d < _SROWS:
                            fwd_v = pltpu.roll(val, _SROWS - d, 0)
                            bwd_v = pltpu.roll(val, d, 0)
                        else:
                            dd = d // SQ
                            fwd_v = pltpu.roll(val, SQ - dd, 1)
                            bwd_v = pltpu.roll(val, dd, 1)
                        pv = jnp.where(side, fwd_v, bwd_v)
                        vals[s] = jnp.where(tp, pv, val)
            d //= 2
        k *= 2
    return keys, vals


def _sort_body(gt_ref, out_ref):
    t_io = lax.broadcasted_iota(jnp.int32, (_SROWS, 1), 0)
    col_io = lax.broadcasted_iota(jnp.int32, (1, SQ), 1)
    t_u = lax.broadcasted_iota(jnp.uint32, (_SROWS, SQ), 0)
    c_u = lax.broadcasted_iota(jnp.uint32, (_SROWS, SQ), 1)
    cnts = [c_u * jnp.uint32(SQ) + t_u + jnp.uint32(s * _SROWS)
            for s in range(_NSLAB)]
    vals = [col_io * SQ + t_io + s * _SROWS for s in range(_NSLAB)]
    keys1 = [_signed_keys(_SK1[0], _SK1[1], c) for c in cnts]
    _, vals = _bitonic_pass(keys1, vals, t_io, col_io)
    keys2 = [_signed_keys(_SK2[0], _SK2[1], c) for c in cnts]
    _, rs = _bitonic_pass(keys2, vals, t_io, col_io)
    packed = [rs[s] * SQ + gt_ref[s * _SROWS:(s + 1) * _SROWS, :]
              for s in range(_NSLAB)]
    packed, _ = _bitonic_pass(packed, None, t_io, col_io)
    gid_t = jnp.concatenate([p & (SQ - 1) for p in packed], axis=0)
    gid = gid_t.T
    gio = lax.broadcasted_iota(jnp.int32, (NGROUPS, 1), 0)
    for a in range(SQ):
        row = gid[a:a + 1, :]
        out_ref[:, a * SQ:(a + 1) * SQ] = (row == gio).astype(jnp.float32)


def kernel(x):
    del x
    g_const = jnp.asarray(np.ascontiguousarray(_group_ids().reshape(SQ, SQ).T))
    return pl.pallas_call(
        _sort_body,
        out_shape=jax.ShapeDtypeStruct((NGROUPS, N), jnp.float32),
    )(g_const)


# speedup vs baseline: 5.9764x; 1.0310x over previous
import numpy as np
import jax
import jax.numpy as jnp
from jax import lax
from jax.experimental import pallas as pl
from jax.experimental.pallas import tpu as pltpu

N = 16384
SQ = 128
_DISTINCT = 64
_MAX_GROUPS = int(1.5 * _DISTINCT)
NGROUPS = min(int(np.random.default_rng(0).integers(1, _MAX_GROUPS + 1)), N)
_BASE, _REM = divmod(N, NGROUPS)

_M32 = 0xFFFFFFFF
_ROTS = ((13, 15, 26, 6), (17, 29, 16, 24))


def _tf2x32_host(k0, k1, x0, x1):
    ks = (k0, k1, (k0 ^ k1 ^ 0x1BD11BDA) & _M32)
    x0 = (x0 + ks[0]) & _M32
    x1 = (x1 + ks[1]) & _M32
    for i in range(5):
        for r in _ROTS[i % 2]:
            x0 = (x0 + x1) & _M32
            x1 = ((x1 << r) | (x1 >> (32 - r))) & _M32
            x1 ^= x0
        x0 = (x0 + ks[(i + 1) % 3]) & _M32
        x1 = (x1 + ks[(i + 2) % 3] + i + 1) & _M32
    return x0, x1


def _derive_round_keys():
    key = _tf2x32_host(0, 0, 0, 1)
    out = []
    for _ in range(2):
        new_key = _tf2x32_host(key[0], key[1], 0, 0)
        subkey = _tf2x32_host(key[0], key[1], 0, 1)
        key = new_key
        out.append(subkey)
    return out


_SK1, _SK2 = _derive_round_keys()


def _group_ids():
    sizes = [_BASE + 1] * _REM + [_BASE] * (NGROUPS - _REM)
    return np.repeat(np.arange(NGROUPS), sizes).astype(np.int32)


def _tf2x32_vec(k0, k1, cnt):
    ks = (jnp.uint32(k0), jnp.uint32(k1),
          jnp.uint32((k0 ^ k1 ^ 0x1BD11BDA) & _M32))
    x0 = jnp.full_like(cnt, ks[0])
    x1 = cnt + ks[1]
    for i in range(5):
        for r in _ROTS[i % 2]:
            x0 = x0 + x1
            x1 = (x1 << jnp.uint32(r)) | (x1 >> jnp.uint32(32 - r))
            x1 = x1 ^ x0
        x0 = x0 + ks[(i + 1) % 3]
        x1 = x1 + ks[(i + 2) % 3] + jnp.uint32(i + 1)
    return x0 ^ x1


def _signed_keys(k0, k1, cnt):
    bits = _tf2x32_vec(k0, k1, cnt)
    return lax.bitcast_convert_type(bits ^ jnp.uint32(0x80000000), jnp.int32)


_NSLAB = 16
_SROWS = SQ // _NSLAB


def _bitonic_pass(keys, vals, t_io, col_io):
    has_val = vals is not None

    def up_of(k, s):
        if k < _SROWS:
            return (t_io & k) == 0
        if k < SQ:
            return ((s * _SROWS) & k) == 0
        if k == N:
            return True
        return (col_io & (k // SQ)) == 0

    k = 2
    while k <= N:
        d = k // 2
        while d >= 1:
            if _SROWS <= d < SQ:
                x = d // _SROWS
                for a in range(_NSLAB):
                    if a & x:
                        continue
                    b = a ^ x
                    up = up_of(k, a)
                    c = keys[b] < keys[a]
                    if isinstance(up, bool):
                        m = c if up else ~c
                    else:
                        m = c == up
                    ka = jnp.where(m, keys[b], keys[a])
                    kb = jnp.where(m, keys[a], keys[b])
                    keys[a], keys[b] = ka, kb
                    if has_val:
                        va = jnp.where(m, vals[b], vals[a])
                        vb = jnp.where(m, vals[a], vals[b])
                        vals[a], vals[b] = va, vb
            else:
                if d < _SROWS:
                    side = (t_io & d) == 0
                    lidx = None
                    sidx = jnp.broadcast_to(t_io ^ d, (_SROWS, SQ))
                else:
                    dd = d // SQ
                    side = (col_io & dd) == 0
                    lidx = jnp.broadcast_to(col_io ^ dd, (_SROWS, SQ))
                up0 = up_of(k, 0)
                ts_mask = None
                if not isinstance(up0, bool):
                    ts_mask = up0 == side

                def _partner(arr):
                    if lidx is not None:
                        return jnp.take_along_axis(arr, lidx, axis=1)
                    return jnp.take_along_axis(arr, sidx, axis=0)

                for s in range(_NSLAB):
                    key = keys[s]
                    pk = _partner(key)
                    up = up_of(k, s)
                    if up is True:
                        tp = (pk < key) == side
                    elif up is False:
                        tp = (pk < key) != side
                    else:
                        tp = (pk < key) == ts_mask
                    keys[s] = jnp.where(tp, pk, key)
                    if has_val:
                        val = vals[s]
                        pv = _partner(val)
                        vals[s] = jnp.where(tp, pv, val)
            d //= 2
        k *= 2
    return keys, vals


def _sort_body(gt_ref, out_ref):
    t_io = lax.broadcasted_iota(jnp.int32, (_SROWS, 1), 0)
    col_io = lax.broadcasted_iota(jnp.int32, (1, SQ), 1)
    t_u = lax.broadcasted_iota(jnp.uint32, (_SROWS, SQ), 0)
    c_u = lax.broadcasted_iota(jnp.uint32, (_SROWS, SQ), 1)
    cnts = [c_u * jnp.uint32(SQ) + t_u + jnp.uint32(s * _SROWS)
            for s in range(_NSLAB)]
    vals = [col_io * SQ + t_io + s * _SROWS for s in range(_NSLAB)]
    keys1 = [_signed_keys(_SK1[0], _SK1[1], c) for c in cnts]
    _, vals = _bitonic_pass(keys1, vals, t_io, col_io)
    keys2 = [_signed_keys(_SK2[0], _SK2[1], c) for c in cnts]
    _, rs = _bitonic_pass(keys2, vals, t_io, col_io)
    packed = [rs[s] * SQ + gt_ref[s * _SROWS:(s + 1) * _SROWS, :]
              for s in range(_NSLAB)]
    packed, _ = _bitonic_pass(packed, None, t_io, col_io)
    gid_t = jnp.concatenate([p & (SQ - 1) for p in packed], axis=0)
    gid = gid_t.T
    gio = lax.broadcasted_iota(jnp.int32, (NGROUPS, 1), 0)
    for a in range(SQ):
        row = gid[a:a + 1, :]
        out_ref[:, a * SQ:(a + 1) * SQ] = (row == gio).astype(jnp.float32)


def kernel(x):
    del x
    g_const = jnp.asarray(np.ascontiguousarray(_group_ids().reshape(SQ, SQ).T))
    return pl.pallas_call(
        _sort_body,
        out_shape=jax.ShapeDtypeStruct((NGROUPS, N), jnp.float32),
    )(g_const)
